# baseline (speedup 1.0000x reference)
import functools

import jax
import jax.numpy as jnp
from jax import lax
from jax.experimental import pallas as pl
from jax.experimental.pallas import tpu as pltpu
from jax.experimental.pallas import tpu_sc as plsc

_BLOCK_SIZES = (50000, 30000, 20000)
_OFF1 = 50000
_OFF2 = 80000
_BASE_DIM = 64
_B = 16384

_NC = 2
_NS = 16
_L = 16
_NW = _NC * _NS
_BPW = _B // _NW
_NSTEP = 4
_ISZ = _BPW // _NSTEP


_V1 = _BLOCK_SIZES[1] + 1
_V2 = _BLOCK_SIZES[2] + 1
_CH1 = 1876
_CH2 = 1251


def _sc_gather_body(ids_hbm, t0, t1, t2, o0, o1, o2,
                    ids_v, idx0, idx1, idx2, e0_v, e1_v, e2_v, s1, s2,
                    sem, sem_stage, sem_sp):
    sid = lax.axis_index("s")
    wid = sid * _NC + lax.axis_index("c")
    base = wid * _BPW
    st1 = jnp.minimum(sid * _CH1, _V1 - _CH1)
    st2 = jnp.minimum(sid * _CH2, _V2 - _CH2)
    c1 = pltpu.async_copy(t1.at[pl.ds(st1, _CH1)], s1.at[pl.ds(st1, _CH1)], sem_stage)
    c2 = pltpu.async_copy(t2.at[pl.ds(st2, _CH2)], s2.at[pl.ds(st2, _CH2)], sem_stage)
    pltpu.sync_copy(ids_hbm.at[pl.ds(base, _BPW)], ids_v)
    for c in range(_BPW // _L):
        v = ids_v[pl.ds(c * _L, _L)]
        in0 = v < _OFF1
        in1 = (v >= _OFF1) & (v < _OFF2)
        in2 = v >= _OFF2
        l0 = jnp.where(in0, v + 1, 0)
        l1 = jnp.where(in1, v - (_OFF1 - 1), 0)
        l2 = jnp.where(in2, v - (_OFF2 - 1), 0)
        r = c // (_ISZ // _L)
        col = (c % (_ISZ // _L)) * _L
        idx0[r, pl.ds(col, _L)] = l0 * 0
        idx1[r, pl.ds(col, _L)] = l1
        idx2[r, pl.ds(col, _L)] = l2
    cps0 = []
    for j in range(_NSTEP):
        cps0.append(pltpu.async_copy(t0.at[idx0.at[j]],
                                     e0_v.at[pl.ds(j * _ISZ, _ISZ)], sem))
    c1.wait()
    c2.wait()
    plsc.subcore_barrier()
    half = _BPW // 2
    for h in range(2):
        cps = []
        for j in range(2):
            dst = pl.ds(j * _ISZ, _ISZ)
            cps.append(pltpu.async_copy(s1.at[idx1.at[2 * h + j]],
                                        e1_v.at[dst], sem_sp))
            cps.append(pltpu.async_copy(s2.at[idx2.at[2 * h + j]],
                                        e2_v.at[dst], sem_sp))
        for cp in cps:
            cp.wait()
        out_h = pl.ds(base + h * half, half)
        pltpu.sync_copy(e1_v, o1.at[out_h])
        pltpu.sync_copy(e2_v, o2.at[out_h])
    for cp in cps0:
        cp.wait()
    pltpu.sync_copy(e0_v, o0.at[pl.ds(base, _BPW)])


def _sc_gather(ids, emb0, emb1, emb2):
    mesh = plsc.VectorSubcoreMesh(core_axis_name="c", subcore_axis_name="s")
    f = functools.partial(
        pl.kernel,
        mesh=mesh,
        out_type=(
            jax.ShapeDtypeStruct((_B, 64), jnp.float32),
            jax.ShapeDtypeStruct((_B, 32), jnp.float32),
            jax.ShapeDtypeStruct((_B, 16), jnp.float32),
        ),
        scratch_types=[
            pltpu.VMEM((_BPW,), jnp.int32),
            pltpu.VMEM((_NSTEP, _ISZ), jnp.int32),
            pltpu.VMEM((_NSTEP, _ISZ), jnp.int32),
            pltpu.VMEM((_NSTEP, _ISZ), jnp.int32),
            pltpu.VMEM((_BPW, 64), jnp.float32),
            pltpu.VMEM((_BPW // 2, 32), jnp.float32),
            pltpu.VMEM((_BPW // 2, 16), jnp.float32),
            pltpu.VMEM_SHARED((_V1, 32), jnp.float32),
            pltpu.VMEM_SHARED((_V2, 16), jnp.float32),
            pltpu.SemaphoreType.DMA,
            pltpu.SemaphoreType.DMA,
            pltpu.SemaphoreType.DMA,
        ],
        compiler_params=pltpu.CompilerParams(use_tc_tiling_on_sc=False),
    )(_sc_gather_body)
    return f(ids, emb0, emb1, emb2)


def _combine_body(ids_ref, e0_ref, e1_ref, e2_ref,
                  W1_ref, b1_ref, W2_ref, b2_ref, out_ref):
    ids = ids_ref[...]
    m0 = (ids < _OFF1).astype(jnp.float32)
    m1 = ((ids >= _OFF1) & (ids < _OFF2)).astype(jnp.float32)
    m2 = (ids >= _OFF2).astype(jnp.float32)
    p1 = jnp.dot(e1_ref[...], W1_ref[...],
                 preferred_element_type=jnp.float32) + b1_ref[...]
    p2 = jnp.dot(e2_ref[...], W2_ref[...],
                 preferred_element_type=jnp.float32) + b2_ref[...]
    out_ref[...] = e0_ref[...] * m0 + p1 * m1 + p2 * m2


def _tc_combine(ids, e0, e1, e2, W1, b1, W2, b2):
    rb = 1024
    grid = (_B // rb,)
    return pl.pallas_call(
        _combine_body,
        grid=grid,
        in_specs=[
            pl.BlockSpec((rb, 1), lambda i: (i, 0)),
            pl.BlockSpec((rb, 64), lambda i: (i, 0)),
            pl.BlockSpec((rb, 32), lambda i: (i, 0)),
            pl.BlockSpec((rb, 16), lambda i: (i, 0)),
            pl.BlockSpec((32, 64), lambda i: (0, 0)),
            pl.BlockSpec((1, 64), lambda i: (0, 0)),
            pl.BlockSpec((16, 64), lambda i: (0, 0)),
            pl.BlockSpec((1, 64), lambda i: (0, 0)),
        ],
        out_specs=pl.BlockSpec((rb, 64), lambda i: (i, 0)),
        out_shape=jax.ShapeDtypeStruct((_B, _BASE_DIM), jnp.float32),
    )(ids.reshape(_B, 1), e0, e1, e2,
      W1, b1.reshape(1, _BASE_DIM), W2, b2.reshape(1, _BASE_DIM))


def kernel(inputs, emb0, emb1, emb2, W1, b1, W2, b2):
    e0, e1, e2 = _sc_gather(inputs, emb0, emb1, emb2)
    return _tc_combine(inputs, e0, e1, e2, W1, b1, W2, b2)


# speedup vs baseline: 3.3336x; 3.3336x over previous
import functools

import jax
import jax.numpy as jnp
from jax import lax
from jax.experimental import pallas as pl
from jax.experimental.pallas import tpu as pltpu
from jax.experimental.pallas import tpu_sc as plsc

_BLOCK_SIZES = (50000, 30000, 20000)
_OFF1 = 50000
_OFF2 = 80000
_BASE_DIM = 64
_B = 16384

_NC = 2
_NS = 16
_L = 16
_NW = _NC * _NS
_BPW = _B // _NW
_NSTEP = 4
_ISZ = _BPW // _NSTEP


_V1 = _BLOCK_SIZES[1] + 1
_V2 = _BLOCK_SIZES[2] + 1
_CH1 = 1876
_CH2 = 1251


def _sc_gather_body(ids_hbm, t0, t1, t2, o0, o1, o2,
                    ids_v, idx0, idx1, idx2, e0_v, e1_v, e2_v, s1, s2,
                    sem, sem_stage, sem_sp):
    sid = lax.axis_index("s")
    wid = sid * _NC + lax.axis_index("c")
    base = wid * _BPW
    st1 = jnp.minimum(sid * _CH1, _V1 - _CH1)
    st2 = jnp.minimum(sid * _CH2, _V2 - _CH2)
    c1 = pltpu.async_copy(t1.at[pl.ds(st1, _CH1)], s1.at[pl.ds(st1, _CH1)], sem_stage)
    c2 = pltpu.async_copy(t2.at[pl.ds(st2, _CH2)], s2.at[pl.ds(st2, _CH2)], sem_stage)
    pltpu.sync_copy(ids_hbm.at[pl.ds(base, _BPW)], ids_v)
    lane = lax.iota(jnp.int32, _L)
    for c in range(_BPW // _L):
        v = ids_v[pl.ds(c * _L, _L)]
        in0 = v < _OFF1
        in1 = (v >= _OFF1) & (v < _OFF2)
        in2 = v >= _OFF2
        dummy = lane + (base + c * _L)
        l0 = jnp.where(in0, v + 1, dummy)
        l1 = jnp.where(in1, v - (_OFF1 - 1), dummy)
        l2 = jnp.where(in2, v - (_OFF2 - 1), dummy)
        r = c // (_ISZ // _L)
        col = (c % (_ISZ // _L)) * _L
        idx0[r, pl.ds(col, _L)] = l0
        idx1[r, pl.ds(col, _L)] = l1
        idx2[r, pl.ds(col, _L)] = l2
    cps0 = []
    for j in range(_NSTEP):
        cps0.append(pltpu.async_copy(t0.at[idx0.at[j]],
                                     e0_v.at[pl.ds(j * _ISZ, _ISZ)], sem))
    c1.wait()
    c2.wait()
    plsc.subcore_barrier()
    half = _BPW // 2
    for h in range(2):
        cps = []
        for j in range(2):
            dst = pl.ds(j * _ISZ, _ISZ)
            cps.append(pltpu.async_copy(s1.at[idx1.at[2 * h + j]],
                                        e1_v.at[dst], sem_sp))
            cps.append(pltpu.async_copy(s2.at[idx2.at[2 * h + j]],
                                        e2_v.at[dst], sem_sp))
        for cp in cps:
            cp.wait()
        out_h = pl.ds(base + h * half, half)
        pltpu.sync_copy(e1_v, o1.at[out_h])
        pltpu.sync_copy(e2_v, o2.at[out_h])
    for cp in cps0:
        cp.wait()
    pltpu.sync_copy(e0_v, o0.at[pl.ds(base, _BPW)])


def _sc_gather(ids, emb0, emb1, emb2):
    mesh = plsc.VectorSubcoreMesh(core_axis_name="c", subcore_axis_name="s")
    f = functools.partial(
        pl.kernel,
        mesh=mesh,
        out_type=(
            jax.ShapeDtypeStruct((_B, 64), jnp.float32),
            jax.ShapeDtypeStruct((_B, 32), jnp.float32),
            jax.ShapeDtypeStruct((_B, 16), jnp.float32),
        ),
        scratch_types=[
            pltpu.VMEM((_BPW,), jnp.int32),
            pltpu.VMEM((_NSTEP, _ISZ), jnp.int32),
            pltpu.VMEM((_NSTEP, _ISZ), jnp.int32),
            pltpu.VMEM((_NSTEP, _ISZ), jnp.int32),
            pltpu.VMEM((_BPW, 64), jnp.float32),
            pltpu.VMEM((_BPW // 2, 32), jnp.float32),
            pltpu.VMEM((_BPW // 2, 16), jnp.float32),
            pltpu.VMEM_SHARED((_V1, 32), jnp.float32),
            pltpu.VMEM_SHARED((_V2, 16), jnp.float32),
            pltpu.SemaphoreType.DMA,
            pltpu.SemaphoreType.DMA,
            pltpu.SemaphoreType.DMA,
        ],
        compiler_params=pltpu.CompilerParams(use_tc_tiling_on_sc=False),
    )(_sc_gather_body)
    return f(ids, emb0, emb1, emb2)


def _combine_body(ids_ref, e0_ref, e1_ref, e2_ref,
                  W1_ref, b1_ref, W2_ref, b2_ref, out_ref):
    ids = ids_ref[...]
    m0 = ids < _OFF1
    m1 = ids < _OFF2
    p1 = jnp.dot(e1_ref[...], W1_ref[...],
                 preferred_element_type=jnp.float32) + b1_ref[...]
    p2 = jnp.dot(e2_ref[...], W2_ref[...],
                 preferred_element_type=jnp.float32) + b2_ref[...]
    out_ref[...] = jnp.where(m0, e0_ref[...], jnp.where(m1, p1, p2))


def _tc_combine(ids, e0, e1, e2, W1, b1, W2, b2):
    rb = 1024
    grid = (_B // rb,)
    return pl.pallas_call(
        _combine_body,
        grid=grid,
        in_specs=[
            pl.BlockSpec((rb, 1), lambda i: (i, 0)),
            pl.BlockSpec((rb, 64), lambda i: (i, 0)),
            pl.BlockSpec((rb, 32), lambda i: (i, 0)),
            pl.BlockSpec((rb, 16), lambda i: (i, 0)),
            pl.BlockSpec((32, 64), lambda i: (0, 0)),
            pl.BlockSpec((1, 64), lambda i: (0, 0)),
            pl.BlockSpec((16, 64), lambda i: (0, 0)),
            pl.BlockSpec((1, 64), lambda i: (0, 0)),
        ],
        out_specs=pl.BlockSpec((rb, 64), lambda i: (i, 0)),
        out_shape=jax.ShapeDtypeStruct((_B, _BASE_DIM), jnp.float32),
    )(ids.reshape(_B, 1), e0, e1, e2,
      W1, b1.reshape(1, _BASE_DIM), W2, b2.reshape(1, _BASE_DIM))


def kernel(inputs, emb0, emb1, emb2, W1, b1, W2, b2):
    e0, e1, e2 = _sc_gather(inputs, emb0, emb1, emb2)
    return _tc_combine(inputs, e0, e1, e2, W1, b1, W2, b2)
